# baseline (speedup 1.0000x reference)
import jax
import jax.numpy as jnp
from jax import lax
from jax.experimental import pallas as pl
from jax.experimental.pallas import tpu as pltpu
from jax.experimental.pallas import tpu_sc as plsc

_NUM_DIM = 10_000_000
_NUM_ACTIVE = 5_000_000
_NC = 2
_NS = 16
_NW = _NC * _NS
_CHUNK = 8000
_NCHUNK = _NUM_ACTIVE // _CHUNK
_GMAX = -(-_NCHUNK // _NW)
_LANES = 16
_UNROLL = 4

_mesh = plsc.VectorSubcoreMesh(core_axis_name="c", subcore_axis_name="s")


@pl.kernel(
    out_type=jax.ShapeDtypeStruct((_NW, _LANES), jnp.float32),
    mesh=_mesh,
    scratch_types=[
        pltpu.VMEM((_CHUNK,), jnp.int32),
        pltpu.VMEM((_CHUNK,), jnp.int32),
        pltpu.VMEM((_CHUNK,), jnp.float32),
        pltpu.VMEM((_CHUNK,), jnp.float32),
        pltpu.VMEM((_LANES,), jnp.float32),
        pltpu.SemaphoreType.DMA,
        pltpu.SemaphoreType.DMA,
        pltpu.SemaphoreType.DMA,
        pltpu.SemaphoreType.DMA,
    ],
)
def _gather_sq_partials(idx_hbm, table_hbm, x0_hbm, out_hbm,
                        idx0, idx1, rows0, rows1, stage,
                        si0, si1, sg0, sg1):
    wid = lax.axis_index("s") * _NC + lax.axis_index("c")
    pltpu.sync_copy(x0_hbm, stage)
    x0 = stage[...]

    idx_b = (idx0, idx1)
    rows_b = (rows0, rows1)
    si = (si0, si1)
    sg = (sg0, sg1)

    n_extra = _NCHUNK % _NW
    n_mine = _NCHUNK // _NW + jnp.where(wid < n_extra, 1, 0)

    def cid(g):
        return jnp.minimum(wid + g * _NW, _NCHUNK - 1)

    def start_idx(g):
        b = g % 2
        return pltpu.async_copy(
            idx_hbm.at[pl.ds(cid(g) * _CHUNK, _CHUNK)], idx_b[b], si[b])

    def start_gather(g):
        b = g % 2
        h = _CHUNK // 2
        d0 = pltpu.async_copy(
            table_hbm.at[idx_b[b].at[pl.ds(0, h)]],
            rows_b[b].at[pl.ds(0, h)], sg[b])
        d1 = pltpu.async_copy(
            table_hbm.at[idx_b[b].at[pl.ds(h, h)]],
            rows_b[b].at[pl.ds(h, h)], sg[b])
        return (d0, d1)

    def reduce_chunk(rows):
        z = jnp.zeros((_LANES,), jnp.float32)

        def inner(i, accs):
            base = i * (_LANES * _UNROLL)
            out = []
            for u in range(_UNROLL):
                v = rows[pl.ds(base + u * _LANES, _LANES)]
                d = v - x0
                out.append(accs[u] + d * d)
            return tuple(out)

        accs = lax.fori_loop(0, _CHUNK // (_LANES * _UNROLL), inner,
                             (z,) * _UNROLL)
        return (accs[0] + accs[1]) + (accs[2] + accs[3])

    c_i0 = start_idx(0)
    c_i1 = start_idx(1)
    c_i0.wait()
    pend_i = c_i1
    pend_g = start_gather(0)

    acc = jnp.zeros((_LANES,), jnp.float32)
    for g in range(_GMAX):
        pend_g[0].wait()
        pend_g[1].wait()
        if g + 1 < _GMAX:
            pend_i.wait()
            next_g = start_gather(g + 1)
        if g + 2 < _GMAX:
            pend_i = start_idx(g + 2)
        csum = reduce_chunk(rows_b[g % 2])
        acc = acc + jnp.where(g < n_mine, csum, jnp.zeros_like(csum))
        if g + 1 < _GMAX:
            pend_g = next_g

    stage[...] = acc
    pltpu.sync_copy(stage, out_hbm.at[wid])


def kernel(parameters_, active_idx, x_0):
    x0_vec = jnp.full((_LANES,), x_0, dtype=jnp.float32)
    partials = _gather_sq_partials(active_idx, parameters_, x0_vec)
    return -jnp.sum(partials)


# speedup vs baseline: 1.0204x; 1.0204x over previous
import jax
import jax.numpy as jnp
from jax import lax
from jax.experimental import pallas as pl
from jax.experimental.pallas import tpu as pltpu
from jax.experimental.pallas import tpu_sc as plsc

_NUM_DIM = 10_000_000
_NUM_ACTIVE = 5_000_000
_NC = 2
_NS = 16
_NW = _NC * _NS
_CHUNK = 8000
_NFULL = 19
_BLEN = 4240
_TLEN = 160
_A_TOTAL = _NFULL * _NW * _CHUNK
_B_TOTAL = _A_TOTAL + _NW * _BLEN
_LENS = [_CHUNK] * _NFULL + [_BLEN, _TLEN]
_GMAX = len(_LENS)
_LANES = 16
_UNROLL = 4

_mesh = plsc.VectorSubcoreMesh(core_axis_name="c", subcore_axis_name="s")


@pl.kernel(
    out_type=jax.ShapeDtypeStruct((_NW, _LANES), jnp.float32),
    mesh=_mesh,
    scratch_types=[
        pltpu.VMEM((_CHUNK,), jnp.int32),
        pltpu.VMEM((_CHUNK,), jnp.int32),
        pltpu.VMEM((_CHUNK,), jnp.float32),
        pltpu.VMEM((_CHUNK,), jnp.float32),
        pltpu.VMEM((_LANES,), jnp.float32),
        pltpu.SemaphoreType.DMA,
        pltpu.SemaphoreType.DMA,
        pltpu.SemaphoreType.DMA,
        pltpu.SemaphoreType.DMA,
    ],
)
def _gather_sq_partials(idx_hbm, table_hbm, x0_hbm, out_hbm,
                        idx0, idx1, rows0, rows1, stage,
                        si0, si1, sg0, sg1):
    wid = lax.axis_index("s") * _NC + lax.axis_index("c")
    pltpu.sync_copy(x0_hbm, stage)
    x0 = stage[...]

    idx_b = (idx0, idx1)
    rows_b = (rows0, rows1)
    si = (si0, si1)
    sg = (sg0, sg1)

    def offset(g):
        if g < _NFULL:
            return (wid + g * _NW) * _CHUNK
        if g == _NFULL:
            return _A_TOTAL + wid * _BLEN
        return _B_TOTAL + jnp.minimum(wid, 1) * _TLEN

    def start_idx(g):
        b = g % 2
        return pltpu.async_copy(
            idx_hbm.at[pl.ds(offset(g), _LENS[g])],
            idx_b[b].at[pl.ds(0, _LENS[g])], si[b])

    def start_gather(g):
        b = g % 2
        return pltpu.async_copy(
            table_hbm.at[idx_b[b].at[pl.ds(0, _LENS[g])]],
            rows_b[b].at[pl.ds(0, _LENS[g])], sg[b])

    def reduce_chunk(rows, n):
        z = jnp.zeros((_LANES,), jnp.float32)
        k4 = n // (_LANES * _UNROLL)

        def inner(i, accs):
            base = i * (_LANES * _UNROLL)
            out = []
            for u in range(_UNROLL):
                v = rows[pl.ds(base + u * _LANES, _LANES)]
                d = v - x0
                out.append(accs[u] + d * d)
            return tuple(out)

        accs = lax.fori_loop(0, k4, inner, (z,) * _UNROLL)
        csum = (accs[0] + accs[1]) + (accs[2] + accs[3])
        for j in range(k4 * _UNROLL, n // _LANES):
            v = rows[pl.ds(j * _LANES, _LANES)]
            d = v - x0
            csum = csum + d * d
        return csum

    c_i0 = start_idx(0)
    c_i1 = start_idx(1)
    c_i0.wait()
    pend_i = c_i1
    pend_g = start_gather(0)

    acc = jnp.zeros((_LANES,), jnp.float32)
    for g in range(_GMAX):
        pend_g.wait()
        if g + 1 < _GMAX:
            pend_i.wait()
            next_g = start_gather(g + 1)
        if g + 2 < _GMAX:
            pend_i = start_idx(g + 2)
        csum = reduce_chunk(rows_b[g % 2], _LENS[g])
        if g == _GMAX - 1:
            csum = jnp.where(wid < 2, csum, jnp.zeros_like(csum))
        acc = acc + csum
        if g + 1 < _GMAX:
            pend_g = next_g

    stage[...] = acc
    pltpu.sync_copy(stage, out_hbm.at[wid])


def kernel(parameters_, active_idx, x_0):
    x0_vec = jnp.full((_LANES,), x_0, dtype=jnp.float32)
    partials = _gather_sq_partials(active_idx, parameters_, x0_vec)
    return -jnp.sum(partials)


# speedup vs baseline: 1.0634x; 1.0421x over previous
import jax
import jax.numpy as jnp
from jax import lax
from jax.experimental import pallas as pl
from jax.experimental.pallas import tpu as pltpu
from jax.experimental.pallas import tpu_sc as plsc

_NUM_DIM = 10_000_000
_NUM_ACTIVE = 5_000_000
_NC = 2
_NS = 16
_NW = _NC * _NS
_CHUNK = 8000
_NFULL = 19
_BLEN = 4240
_TLEN = 160
_A_TOTAL = _NFULL * _NW * _CHUNK
_B_TOTAL = _A_TOTAL + _NW * _BLEN
_LENS = [_CHUNK] * _NFULL + [_BLEN, _TLEN]
_GMAX = len(_LENS)
_LANES = 16
_UNROLL = 4

_mesh = plsc.VectorSubcoreMesh(core_axis_name="c", subcore_axis_name="s")


@pl.kernel(
    out_type=jax.ShapeDtypeStruct((_NW, _LANES), jnp.float32),
    mesh=_mesh,
    scratch_types=[
        pltpu.VMEM((_CHUNK,), jnp.int32),
        pltpu.VMEM((_CHUNK,), jnp.int32),
        pltpu.VMEM((_CHUNK,), jnp.float32),
        pltpu.VMEM((_CHUNK,), jnp.float32),
        pltpu.VMEM((_LANES,), jnp.float32),
        pltpu.SemaphoreType.DMA,
        pltpu.SemaphoreType.DMA,
        pltpu.SemaphoreType.DMA,
        pltpu.SemaphoreType.DMA,
    ],
)
def _gather_sq_partials(idx_hbm, table_hbm, x0_hbm, out_hbm,
                        idx0, idx1, rows0, rows1, stage,
                        si0, si1, sg0, sg1):
    wid = lax.axis_index("s") * _NC + lax.axis_index("c")
    pltpu.sync_copy(x0_hbm, stage.at[pl.ds(0, 1)])
    x0 = jnp.full((_LANES,), stage[...][0], dtype=jnp.float32)

    idx_b = (idx0, idx1)
    rows_b = (rows0, rows1)
    si = (si0, si1)
    sg = (sg0, sg1)

    def offset(g):
        if g < _NFULL:
            return (wid + g * _NW) * _CHUNK
        if g == _NFULL:
            return _A_TOTAL + wid * _BLEN
        return _B_TOTAL + jnp.minimum(wid, 1) * _TLEN

    def start_idx(g):
        b = g % 2
        return pltpu.async_copy(
            idx_hbm.at[pl.ds(offset(g), _LENS[g])],
            idx_b[b].at[pl.ds(0, _LENS[g])], si[b])

    def start_gather(g):
        b = g % 2
        return pltpu.async_copy(
            table_hbm.at[idx_b[b].at[pl.ds(0, _LENS[g])]],
            rows_b[b].at[pl.ds(0, _LENS[g])], sg[b])

    def reduce_chunk(rows, n):
        z = jnp.zeros((_LANES,), jnp.float32)
        k4 = n // (_LANES * _UNROLL)

        def inner(i, accs):
            base = i * (_LANES * _UNROLL)
            out = []
            for u in range(_UNROLL):
                v = rows[pl.ds(base + u * _LANES, _LANES)]
                d = v - x0
                out.append(accs[u] + d * d)
            return tuple(out)

        accs = lax.fori_loop(0, k4, inner, (z,) * _UNROLL)
        csum = (accs[0] + accs[1]) + (accs[2] + accs[3])
        for j in range(k4 * _UNROLL, n // _LANES):
            v = rows[pl.ds(j * _LANES, _LANES)]
            d = v - x0
            csum = csum + d * d
        return csum

    c_i0 = start_idx(0)
    c_i1 = start_idx(1)
    c_i0.wait()
    pend_i = c_i1
    pend_g = start_gather(0)

    acc = jnp.zeros((_LANES,), jnp.float32)
    for g in range(_GMAX):
        if g + 1 < _GMAX:
            pend_i.wait()
            next_g = start_gather(g + 1)
        pend_g.wait()
        if g + 2 < _GMAX:
            pend_i = start_idx(g + 2)
        csum = reduce_chunk(rows_b[g % 2], _LENS[g])
        if g == _GMAX - 1:
            csum = jnp.where(wid < 2, csum, jnp.zeros_like(csum))
        acc = acc + csum
        if g + 1 < _GMAX:
            pend_g = next_g

    stage[...] = acc
    pltpu.sync_copy(stage, out_hbm.at[wid])


def kernel(parameters_, active_idx, x_0):
    x0_arr = jnp.reshape(x_0.astype(jnp.float32), (1,))
    partials = _gather_sq_partials(active_idx, parameters_, x0_arr)
    return -jnp.sum(partials)
